# baseline (speedup 1.0000x reference)
import functools

import jax
import jax.numpy as jnp
import numpy as np
from jax import lax
from jax.experimental import pallas as pl
from jax.experimental.pallas import tpu as pltpu
from jax.experimental.pallas import tpu_sc as plsc

BATCH = 16384
EMB_DIM = 256
OTHER_DIM = 32
HIDDEN = 64
OUT_DIM = 64
IN1 = 2 * EMB_DIM + OTHER_DIM

NC = 2
NS = 16
NW = NC * NS
CHUNK = 128

NCH = 2
CB = BATCH // NCH
NIDX_C = 2 * CB
BPW = NIDX_C // NW
NCHUNK = BPW // CHUNK
L = 16

BB = 1024
NB = CB // BB


def _make_sc_gather(chunk_start: int):
    mesh = plsc.VectorSubcoreMesh(core_axis_name="c", subcore_axis_name="s")

    @functools.partial(
        pl.kernel,
        mesh=mesh,
        out_type=jax.ShapeDtypeStruct((NIDX_C, EMB_DIM // 2), jnp.int32),
        scratch_types=[
            pltpu.VMEM((BPW,), jnp.float32),
            pltpu.VMEM((BPW,), jnp.int32),
            pltpu.VMEM((CHUNK, EMB_DIM), jnp.float32),
            pltpu.VMEM((CHUNK, EMB_DIM), jnp.float32),
            pltpu.VMEM((CHUNK, EMB_DIM // 2), jnp.int32),
            pltpu.VMEM((CHUNK, EMB_DIM // 2), jnp.int32),
            pltpu.SemaphoreType.DMA,
            pltpu.SemaphoreType.DMA,
        ],
    )
    def gather_k(table_hbm, xt_hbm, out_hbm, idx_f, idx_v, r0, r1, p0, p1, sem_g, sem_o):
        rows, packed = (r0, r1), (p0, p1)
        wid = lax.axis_index("s") * NC + lax.axis_index("c")
        base = wid * BPW
        id_row = base // CB
        src_off = chunk_start + base - id_row * CB

        pltpu.sync_copy(xt_hbm.at[id_row, pl.ds(src_off, BPW)], idx_f)

        def conv(k, carry):
            idx_v[pl.ds(k * L, L)] = idx_f[pl.ds(k * L, L)].astype(jnp.int32)
            return carry

        lax.fori_loop(0, BPW // L, conv, 0)

        hi_mask = jnp.full((L,), -65536, jnp.int32)
        sh16 = jnp.full((L,), 16, jnp.int32)

        def to_bf16(src, dst):
            def row(r, carry):
                for g in range(EMB_DIM // 32):
                    a = lax.bitcast_convert_type(src[r, pl.ds(g * 32, L)], jnp.int32)
                    b = lax.bitcast_convert_type(src[r, pl.ds(g * 32 + L, L)], jnp.int32)
                    w = lax.shift_right_logical(a, sh16) | (b & hi_mask)
                    dst[r, pl.ds(g * L, L)] = w
                return carry

            lax.fori_loop(0, CHUNK, row, 0)

        gathers = [None, None]
        outs = [None, None]
        for j in range(NCHUNK):
            b = j % 2
            gathers[b] = pltpu.async_copy(
                table_hbm.at[idx_v.at[pl.ds(j * CHUNK, CHUNK)]], rows[b], sem_g
            )
            if j >= 1:
                bp = (j - 1) % 2
                gathers[bp].wait()
                if j >= 3:
                    outs[bp].wait()
                to_bf16(rows[bp], packed[bp])
                outs[bp] = pltpu.async_copy(
                    packed[bp], out_hbm.at[pl.ds(base + (j - 1) * CHUNK, CHUNK)], sem_o
                )
        bl = (NCHUNK - 1) % 2
        gathers[bl].wait()
        if NCHUNK >= 3:
            outs[bl].wait()
        to_bf16(rows[bl], packed[bl])
        outs[bl] = pltpu.async_copy(
            packed[bl], out_hbm.at[pl.ds(base + (NCHUNK - 1) * CHUNK, CHUNK)], sem_o
        )
        if NCHUNK >= 2:
            outs[1 - bl].wait()
        outs[bl].wait()

    return gather_k


def _mlp_body(g1, g2, ot, w1t, b1, w2, b2, w3, b3, obuf, out):
    del obuf
    w1full = w1t[...]
    w1at = w1full[:, :EMB_DIM].astype(jnp.bfloat16)
    w1bt = w1full[:, EMB_DIM : 2 * EMB_DIM].astype(jnp.bfloat16)
    w1ct = w1full[:, 2 * EMB_DIM :]
    dn_rt = (((1,), (1,)), ((), ()))
    dn_lt = (((0,), (0,)), ((), ()))
    hg = (
        lax.dot_general(g1[...], w1at, dn_rt, preferred_element_type=jnp.float32)
        + lax.dot_general(g2[...], w1bt, dn_rt, preferred_element_type=jnp.float32)
        + b1[...]
    )
    h1t = hg.T + jnp.dot(w1ct, ot[...], preferred_element_type=jnp.float32)
    h1t = jnp.maximum(h1t, 0.0)
    h2t = jnp.maximum(
        lax.dot_general(w2[...], h1t, dn_lt, preferred_element_type=jnp.float32)
        + b2[...].T,
        0.0,
    )
    out[...] = (
        lax.dot_general(w3[...], h2t, dn_lt, preferred_element_type=jnp.float32)
        + b3[...].T
    )


def _mlp_call(c, gathered, other_t, W1t, b1, W2, b2, W3, b3, obuf=None):
    full = lambda shape: pl.BlockSpec(shape, lambda i: (0, 0))
    in_specs = [
        pl.BlockSpec((BB, EMB_DIM), lambda i: (i, 0)),
        pl.BlockSpec((BB, EMB_DIM), lambda i: (i + NB, 0)),
        pl.BlockSpec((OTHER_DIM, BB), lambda i: (0, i + c * NB)),
        full((HIDDEN, IN1)),
        full((1, HIDDEN)),
        full((HIDDEN, HIDDEN)),
        full((1, HIDDEN)),
        full((HIDDEN, OUT_DIM)),
        full((1, OUT_DIM)),
    ]
    args = [gathered, gathered, other_t, W1t, b1, W2, b2, W3, b3]
    aliases = {}
    if obuf is not None:
        in_specs.append(pl.BlockSpec(memory_space=pl.ANY))
        args.append(obuf)
        aliases = {9: 0}
        body = _mlp_body
    else:

        def body(*refs):
            _mlp_body(*refs[:9], None, refs[9])

    return pl.pallas_call(
        body,
        grid=(NB,),
        in_specs=in_specs,
        out_specs=pl.BlockSpec((OUT_DIM, BB), lambda i: (0, i + c * NB)),
        out_shape=jax.ShapeDtypeStruct((OUT_DIM, BATCH), jnp.float32),
        input_output_aliases=aliases,
    )(*args)


_pg = np.arange(16)
_perm32 = np.stack([_pg, _pg + 16], axis=1).reshape(-1)
_PERM = np.concatenate(
    [g * 32 + _perm32 for g in range(2 * EMB_DIM // 32)] + [np.arange(2 * EMB_DIM, IN1)]
).astype(np.int32)


def kernel(x, emb, W1, b1, W2, b2, W3, b3):
    x_t = x.T
    other_t = x_t[2:]

    W1t = W1.T[:, _PERM]
    b1r = b1.reshape(1, HIDDEN)
    b2r = b2.reshape(1, HIDDEN)
    b3r = b3.reshape(1, OUT_DIM)

    obuf = None
    for c in range(NCH):
        packed = _make_sc_gather(c * CB)(emb, x_t)
        gathered = lax.bitcast_convert_type(packed, jnp.bfloat16).reshape(
            NIDX_C, EMB_DIM
        )
        obuf = _mlp_call(c, gathered, other_t, W1t, b1r, W2, b2r, W3, b3r, obuf)
    return obuf.T


# speedup vs baseline: 3.2319x; 3.2319x over previous
import functools

import jax
import jax.numpy as jnp
from jax import lax
from jax.experimental import pallas as pl
from jax.experimental.pallas import tpu as pltpu
from jax.experimental.pallas import tpu_sc as plsc

BATCH = 16384
EMB_DIM = 256
OTHER_DIM = 32
HIDDEN = 64
OUT_DIM = 64
IN1 = 2 * EMB_DIM + OTHER_DIM

NC = 2
NS = 16
NW = NC * NS
CHUNK = 128

CHUNKS = (4096, 8192, 4096)
L = 16

BB = 1024


def _make_sc_gather(chunk_start: int, cb: int):
    mesh = plsc.VectorSubcoreMesh(core_axis_name="c", subcore_axis_name="s")
    nidx_c = 2 * cb
    BPW = nidx_c // NW
    NCHUNK = BPW // CHUNK
    nbuf = min(3, NCHUNK)

    @functools.partial(
        pl.kernel,
        mesh=mesh,
        out_type=jax.ShapeDtypeStruct((nidx_c, EMB_DIM), jnp.float32),
        scratch_types=[
            pltpu.VMEM((BPW,), jnp.float32),
            pltpu.VMEM((BPW,), jnp.int32),
        ] + [pltpu.VMEM((CHUNK, EMB_DIM), jnp.float32) for _ in range(nbuf)] + [
            pltpu.SemaphoreType.DMA,
            pltpu.SemaphoreType.DMA,
        ],
    )
    def gather_k(table_hbm, xt_hbm, out_hbm, idx_f, idx_v, *rest):
        rows, (sem_g, sem_o) = rest[:nbuf], rest[nbuf:]
        wid = lax.axis_index("s") * NC + lax.axis_index("c")
        base = wid * BPW
        id_row = base // cb
        src_off = chunk_start + base - id_row * cb

        pltpu.sync_copy(xt_hbm.at[id_row, pl.ds(src_off, BPW)], idx_f)

        def conv(k, carry):
            idx_v[pl.ds(k * L, L)] = idx_f[pl.ds(k * L, L)].astype(jnp.int32)
            return carry

        lax.fori_loop(0, BPW // L, conv, 0)

        gathers = [None] * nbuf
        outs = [None] * nbuf
        for j in range(NCHUNK):
            b = j % nbuf
            if j >= nbuf:
                outs[b].wait()
            gathers[b] = pltpu.async_copy(
                table_hbm.at[idx_v.at[pl.ds(j * CHUNK, CHUNK)]], rows[b], sem_g
            )
            if j >= 1:
                bp = (j - 1) % nbuf
                gathers[bp].wait()
                outs[bp] = pltpu.async_copy(
                    rows[bp], out_hbm.at[pl.ds(base + (j - 1) * CHUNK, CHUNK)], sem_o
                )
        bl = (NCHUNK - 1) % nbuf
        gathers[bl].wait()
        outs[bl] = pltpu.async_copy(
            rows[bl], out_hbm.at[pl.ds(base + (NCHUNK - 1) * CHUNK, CHUNK)], sem_o
        )
        for c in range(max(0, NCHUNK - nbuf), NCHUNK):
            outs[c % nbuf].wait()

    return gather_k


def _mlp_body(g1, g2, ot, w1t, b1, w2, b2, w3, b3, obuf, out):
    del obuf
    w1full = w1t[...]
    w1at = w1full[:, :EMB_DIM]
    w1bt = w1full[:, EMB_DIM : 2 * EMB_DIM]
    w1ct = w1full[:, 2 * EMB_DIM :]
    dn_rt = (((1,), (1,)), ((), ()))
    dn_lt = (((0,), (0,)), ((), ()))
    hg = (
        lax.dot_general(g1[...], w1at, dn_rt, preferred_element_type=jnp.float32)
        + lax.dot_general(g2[...], w1bt, dn_rt, preferred_element_type=jnp.float32)
        + b1[...]
    )
    h1t = hg.T + jnp.dot(w1ct, ot[...], preferred_element_type=jnp.float32)
    h1t = jnp.maximum(h1t, 0.0)
    h2t = jnp.maximum(
        lax.dot_general(w2[...], h1t, dn_lt, preferred_element_type=jnp.float32)
        + b2[...].T,
        0.0,
    )
    out[...] = (
        lax.dot_general(w3[...], h2t, dn_lt, preferred_element_type=jnp.float32)
        + b3[...].T
    )


def _mlp_call(block0, cb, gathered, other_t, W1t, b1, W2, b2, W3, b3, obuf=None):
    nb = cb // BB
    full = lambda shape: pl.BlockSpec(shape, lambda i: (0, 0))
    in_specs = [
        pl.BlockSpec((BB, EMB_DIM), lambda i: (i, 0)),
        pl.BlockSpec((BB, EMB_DIM), lambda i: (i + nb, 0)),
        pl.BlockSpec((OTHER_DIM, BB), lambda i: (0, i + block0)),
        full((HIDDEN, IN1)),
        full((1, HIDDEN)),
        full((HIDDEN, HIDDEN)),
        full((1, HIDDEN)),
        full((HIDDEN, OUT_DIM)),
        full((1, OUT_DIM)),
    ]
    args = [gathered, gathered, other_t, W1t, b1, W2, b2, W3, b3]
    aliases = {}
    if obuf is not None:
        in_specs.append(pl.BlockSpec(memory_space=pl.ANY))
        args.append(obuf)
        aliases = {9: 0}
        body = _mlp_body
    else:

        def body(*refs):
            _mlp_body(*refs[:9], None, refs[9])

    return pl.pallas_call(
        body,
        grid=(nb,),
        in_specs=in_specs,
        out_specs=pl.BlockSpec((OUT_DIM, BB), lambda i: (0, i + block0)),
        out_shape=jax.ShapeDtypeStruct((OUT_DIM, BATCH), jnp.float32),
        input_output_aliases=aliases,
    )(*args)


def kernel(x, emb, W1, b1, W2, b2, W3, b3):
    x_t = x.T
    other_t = x_t[2:]

    W1t = W1.T
    b1r = b1.reshape(1, HIDDEN)
    b2r = b2.reshape(1, HIDDEN)
    b3r = b3.reshape(1, OUT_DIM)

    obuf = None
    start = 0
    for cb in CHUNKS:
        gathered = _make_sc_gather(start, cb)(emb, x_t)
        obuf = _mlp_call(
            start // BB, cb, gathered, other_t, W1t, b1r, W2, b2r, W3, b3r, obuf
        )
        start += cb
    return obuf.T


# speedup vs baseline: 3.2523x; 1.0063x over previous
import functools

import jax
import jax.numpy as jnp
from jax import lax
from jax.experimental import pallas as pl
from jax.experimental.pallas import tpu as pltpu
from jax.experimental.pallas import tpu_sc as plsc

BATCH = 16384
EMB_DIM = 256
OTHER_DIM = 32
HIDDEN = 64
OUT_DIM = 64
IN1 = 2 * EMB_DIM + OTHER_DIM

NC = 2
NS = 16
NW = NC * NS
CHUNK = 128

CHUNKS = (12288, 4096)
L = 16

BB = 1024


def _make_sc_gather(chunk_start: int, cb: int):
    mesh = plsc.VectorSubcoreMesh(core_axis_name="c", subcore_axis_name="s")
    nidx_c = 2 * cb
    BPW = nidx_c // NW
    NCHUNK = BPW // CHUNK
    nbuf = min(3, NCHUNK)

    @functools.partial(
        pl.kernel,
        mesh=mesh,
        out_type=jax.ShapeDtypeStruct((nidx_c, EMB_DIM), jnp.float32),
        scratch_types=[
            pltpu.VMEM((BPW,), jnp.float32),
            pltpu.VMEM((BPW,), jnp.int32),
        ] + [pltpu.VMEM((CHUNK, EMB_DIM), jnp.float32) for _ in range(nbuf)] + [
            pltpu.SemaphoreType.DMA,
            pltpu.SemaphoreType.DMA,
        ],
    )
    def gather_k(table_hbm, xt_hbm, out_hbm, idx_f, idx_v, *rest):
        rows, (sem_g, sem_o) = rest[:nbuf], rest[nbuf:]
        wid = lax.axis_index("s") * NC + lax.axis_index("c")
        base = wid * BPW
        id_row = base // cb
        src_off = chunk_start + base - id_row * cb

        pltpu.sync_copy(xt_hbm.at[id_row, pl.ds(src_off, BPW)], idx_f)

        def conv(k, carry):
            idx_v[pl.ds(k * L, L)] = idx_f[pl.ds(k * L, L)].astype(jnp.int32)
            return carry

        lax.fori_loop(0, BPW // L, conv, 0)

        gathers = [None] * nbuf
        outs = [None] * nbuf
        for j in range(NCHUNK):
            b = j % nbuf
            if j >= nbuf:
                outs[b].wait()
            gathers[b] = pltpu.async_copy(
                table_hbm.at[idx_v.at[pl.ds(j * CHUNK, CHUNK)]], rows[b], sem_g
            )
            if j >= 1:
                bp = (j - 1) % nbuf
                gathers[bp].wait()
                outs[bp] = pltpu.async_copy(
                    rows[bp], out_hbm.at[pl.ds(base + (j - 1) * CHUNK, CHUNK)], sem_o
                )
        bl = (NCHUNK - 1) % nbuf
        gathers[bl].wait()
        outs[bl] = pltpu.async_copy(
            rows[bl], out_hbm.at[pl.ds(base + (NCHUNK - 1) * CHUNK, CHUNK)], sem_o
        )
        for c in range(max(0, NCHUNK - nbuf), NCHUNK):
            outs[c % nbuf].wait()

    return gather_k


def _mlp_body(g1, g2, ot, w1t, b1, w2, b2, w3, b3, obuf, out):
    del obuf
    w1full = w1t[...]
    w1at = w1full[:, :EMB_DIM]
    w1bt = w1full[:, EMB_DIM : 2 * EMB_DIM]
    w1ct = w1full[:, 2 * EMB_DIM :]
    dn_rt = (((1,), (1,)), ((), ()))
    dn_lt = (((0,), (0,)), ((), ()))
    hg = (
        lax.dot_general(g1[...], w1at, dn_rt, preferred_element_type=jnp.float32)
        + lax.dot_general(g2[...], w1bt, dn_rt, preferred_element_type=jnp.float32)
        + b1[...]
    )
    h1t = hg.T + jnp.dot(w1ct, ot[...], preferred_element_type=jnp.float32)
    h1t = jnp.maximum(h1t, 0.0)
    h2t = jnp.maximum(
        lax.dot_general(w2[...], h1t, dn_lt, preferred_element_type=jnp.float32)
        + b2[...].T,
        0.0,
    )
    out[...] = (
        lax.dot_general(w3[...], h2t, dn_lt, preferred_element_type=jnp.float32)
        + b3[...].T
    )


def _mlp_call(block0, cb, gathered, other_t, W1t, b1, W2, b2, W3, b3, obuf=None):
    nb = cb // BB
    full = lambda shape: pl.BlockSpec(shape, lambda i: (0, 0))
    in_specs = [
        pl.BlockSpec((BB, EMB_DIM), lambda i: (i, 0)),
        pl.BlockSpec((BB, EMB_DIM), lambda i: (i + nb, 0)),
        pl.BlockSpec((OTHER_DIM, BB), lambda i: (0, i + block0)),
        full((HIDDEN, IN1)),
        full((1, HIDDEN)),
        full((HIDDEN, HIDDEN)),
        full((1, HIDDEN)),
        full((HIDDEN, OUT_DIM)),
        full((1, OUT_DIM)),
    ]
    args = [gathered, gathered, other_t, W1t, b1, W2, b2, W3, b3]
    aliases = {}
    if obuf is not None:
        in_specs.append(pl.BlockSpec(memory_space=pl.ANY))
        args.append(obuf)
        aliases = {9: 0}
        body = _mlp_body
    else:

        def body(*refs):
            _mlp_body(*refs[:9], None, refs[9])

    return pl.pallas_call(
        body,
        grid=(nb,),
        in_specs=in_specs,
        out_specs=pl.BlockSpec((OUT_DIM, BB), lambda i: (0, i + block0)),
        out_shape=jax.ShapeDtypeStruct((OUT_DIM, BATCH), jnp.float32),
        input_output_aliases=aliases,
    )(*args)


def kernel(x, emb, W1, b1, W2, b2, W3, b3):
    x_t = x.T
    other_t = x_t[2:]

    W1t = W1.T
    b1r = b1.reshape(1, HIDDEN)
    b2r = b2.reshape(1, HIDDEN)
    b3r = b3.reshape(1, OUT_DIM)

    obuf = None
    start = 0
    for cb in CHUNKS:
        gathered = _make_sc_gather(start, cb)(emb, x_t)
        obuf = _mlp_call(
            start // BB, cb, gathered, other_t, W1t, b1r, W2, b2r, W3, b3r, obuf
        )
        start += cb
    return obuf.T


# speedup vs baseline: 3.5038x; 1.0773x over previous
import functools

import jax
import jax.numpy as jnp
from jax import lax
from jax.experimental import pallas as pl
from jax.experimental.pallas import tpu as pltpu
from jax.experimental.pallas import tpu_sc as plsc

BATCH = 16384
EMB_DIM = 256
OTHER_DIM = 32
HIDDEN = 64
OUT_DIM = 64
IN1 = 2 * EMB_DIM + OTHER_DIM

NC = 2
NS = 16
NW = NC * NS
CHUNK = 128

CHUNKS = (8192, 8192)
L = 16

BB = 2048


def _make_sc_gather(chunk_start: int, cb: int):
    mesh = plsc.VectorSubcoreMesh(core_axis_name="c", subcore_axis_name="s")
    nidx_c = 2 * cb
    BPW = nidx_c // NW
    NCHUNK = BPW // CHUNK
    nbuf = min(3, NCHUNK)

    @functools.partial(
        pl.kernel,
        mesh=mesh,
        out_type=jax.ShapeDtypeStruct((nidx_c, EMB_DIM), jnp.float32),
        scratch_types=[
            pltpu.VMEM((BPW,), jnp.float32),
            pltpu.VMEM((BPW,), jnp.int32),
        ] + [pltpu.VMEM((CHUNK, EMB_DIM), jnp.float32) for _ in range(nbuf)] + [
            pltpu.SemaphoreType.DMA,
            pltpu.SemaphoreType.DMA,
        ],
    )
    def gather_k(table_hbm, xt_hbm, out_hbm, idx_f, idx_v, *rest):
        rows, (sem_g, sem_o) = rest[:nbuf], rest[nbuf:]
        wid = lax.axis_index("s") * NC + lax.axis_index("c")
        base = wid * BPW
        id_row = base // cb
        src_off = chunk_start + base - id_row * cb

        pltpu.sync_copy(xt_hbm.at[id_row, pl.ds(src_off, BPW)], idx_f)

        def conv(k, carry):
            idx_v[pl.ds(k * L, L)] = idx_f[pl.ds(k * L, L)].astype(jnp.int32)
            return carry

        lax.fori_loop(0, BPW // L, conv, 0)

        gathers = [None] * nbuf
        outs = [None] * nbuf
        for j in range(NCHUNK):
            b = j % nbuf
            if j >= nbuf:
                outs[b].wait()
            gathers[b] = pltpu.async_copy(
                table_hbm.at[idx_v.at[pl.ds(j * CHUNK, CHUNK)]], rows[b], sem_g
            )
            if j >= 1:
                bp = (j - 1) % nbuf
                gathers[bp].wait()
                outs[bp] = pltpu.async_copy(
                    rows[bp], out_hbm.at[pl.ds(base + (j - 1) * CHUNK, CHUNK)], sem_o
                )
        bl = (NCHUNK - 1) % nbuf
        gathers[bl].wait()
        outs[bl] = pltpu.async_copy(
            rows[bl], out_hbm.at[pl.ds(base + (NCHUNK - 1) * CHUNK, CHUNK)], sem_o
        )
        for c in range(max(0, NCHUNK - nbuf), NCHUNK):
            outs[c % nbuf].wait()

    return gather_k


def _mlp_body(g1, g2, ot, w1t, b1, w2, b2, w3, b3, obuf, out):
    del obuf
    w1full = w1t[...]
    w1at = w1full[:, :EMB_DIM]
    w1bt = w1full[:, EMB_DIM : 2 * EMB_DIM]
    w1ct = w1full[:, 2 * EMB_DIM :]
    dn_rt = (((1,), (1,)), ((), ()))
    dn_lt = (((0,), (0,)), ((), ()))
    hg = (
        lax.dot_general(g1[...], w1at, dn_rt, preferred_element_type=jnp.float32)
        + lax.dot_general(g2[...], w1bt, dn_rt, preferred_element_type=jnp.float32)
        + b1[...]
    )
    h1t = hg.T + jnp.dot(w1ct, ot[...], preferred_element_type=jnp.float32)
    h1t = jnp.maximum(h1t, 0.0)
    h2t = jnp.maximum(
        lax.dot_general(w2[...], h1t, dn_lt, preferred_element_type=jnp.float32)
        + b2[...].T,
        0.0,
    )
    out[...] = (
        lax.dot_general(w3[...], h2t, dn_lt, preferred_element_type=jnp.float32)
        + b3[...].T
    )


def _mlp_call(block0, cb, gathered, other_t, W1t, b1, W2, b2, W3, b3, obuf=None):
    nb = cb // BB
    full = lambda shape: pl.BlockSpec(shape, lambda i: (0, 0))
    in_specs = [
        pl.BlockSpec((BB, EMB_DIM), lambda i: (i, 0)),
        pl.BlockSpec((BB, EMB_DIM), lambda i: (i + nb, 0)),
        pl.BlockSpec((OTHER_DIM, BB), lambda i: (0, i + block0)),
        full((HIDDEN, IN1)),
        full((1, HIDDEN)),
        full((HIDDEN, HIDDEN)),
        full((1, HIDDEN)),
        full((HIDDEN, OUT_DIM)),
        full((1, OUT_DIM)),
    ]
    args = [gathered, gathered, other_t, W1t, b1, W2, b2, W3, b3]
    aliases = {}
    if obuf is not None:
        in_specs.append(pl.BlockSpec(memory_space=pl.ANY))
        args.append(obuf)
        aliases = {9: 0}
        body = _mlp_body
    else:

        def body(*refs):
            _mlp_body(*refs[:9], None, refs[9])

    return pl.pallas_call(
        body,
        grid=(nb,),
        in_specs=in_specs,
        out_specs=pl.BlockSpec((OUT_DIM, BB), lambda i: (0, i + block0)),
        out_shape=jax.ShapeDtypeStruct((OUT_DIM, BATCH), jnp.float32),
        input_output_aliases=aliases,
    )(*args)


def kernel(x, emb, W1, b1, W2, b2, W3, b3):
    x_t = x.T
    other_t = x_t[2:]

    W1t = W1.T
    b1r = b1.reshape(1, HIDDEN)
    b2r = b2.reshape(1, HIDDEN)
    b3r = b3.reshape(1, OUT_DIM)

    obuf = None
    start = 0
    for cb in CHUNKS:
        gathered = _make_sc_gather(start, cb)(emb, x_t)
        obuf = _mlp_call(
            start // BB, cb, gathered, other_t, W1t, b1r, W2, b2r, W3, b3r, obuf
        )
        start += cb
    return obuf.T


# speedup vs baseline: 3.5365x; 1.0093x over previous
import functools

import jax
import jax.numpy as jnp
from jax import lax
from jax.experimental import pallas as pl
from jax.experimental.pallas import tpu as pltpu
from jax.experimental.pallas import tpu_sc as plsc

BATCH = 16384
EMB_DIM = 256
OTHER_DIM = 32
HIDDEN = 64
OUT_DIM = 64
IN1 = 2 * EMB_DIM + OTHER_DIM

NC = 2
NS = 16
NW = NC * NS
CHUNK = 128

CHUNKS = (8192, 8192)
L = 16

BB = 4096


def _make_sc_gather(chunk_start: int, cb: int):
    mesh = plsc.VectorSubcoreMesh(core_axis_name="c", subcore_axis_name="s")
    nidx_c = 2 * cb
    BPW = nidx_c // NW
    NCHUNK = BPW // CHUNK
    nbuf = min(3, NCHUNK)

    @functools.partial(
        pl.kernel,
        mesh=mesh,
        out_type=jax.ShapeDtypeStruct((nidx_c, EMB_DIM), jnp.float32),
        scratch_types=[
            pltpu.VMEM((BPW,), jnp.float32),
            pltpu.VMEM((BPW,), jnp.int32),
        ] + [pltpu.VMEM((CHUNK, EMB_DIM), jnp.float32) for _ in range(nbuf)] + [
            pltpu.SemaphoreType.DMA,
            pltpu.SemaphoreType.DMA,
        ],
    )
    def gather_k(table_hbm, xt_hbm, out_hbm, idx_f, idx_v, *rest):
        rows, (sem_g, sem_o) = rest[:nbuf], rest[nbuf:]
        wid = lax.axis_index("s") * NC + lax.axis_index("c")
        base = wid * BPW
        id_row = base // cb
        src_off = chunk_start + base - id_row * cb

        pltpu.sync_copy(xt_hbm.at[id_row, pl.ds(src_off, BPW)], idx_f)

        def conv(k, carry):
            idx_v[pl.ds(k * L, L)] = idx_f[pl.ds(k * L, L)].astype(jnp.int32)
            return carry

        lax.fori_loop(0, BPW // L, conv, 0)

        gathers = [None] * nbuf
        outs = [None] * nbuf
        for j in range(NCHUNK):
            b = j % nbuf
            if j >= nbuf:
                outs[b].wait()
            gathers[b] = pltpu.async_copy(
                table_hbm.at[idx_v.at[pl.ds(j * CHUNK, CHUNK)]], rows[b], sem_g
            )
            if j >= 1:
                bp = (j - 1) % nbuf
                gathers[bp].wait()
                outs[bp] = pltpu.async_copy(
                    rows[bp], out_hbm.at[pl.ds(base + (j - 1) * CHUNK, CHUNK)], sem_o
                )
        bl = (NCHUNK - 1) % nbuf
        gathers[bl].wait()
        outs[bl] = pltpu.async_copy(
            rows[bl], out_hbm.at[pl.ds(base + (NCHUNK - 1) * CHUNK, CHUNK)], sem_o
        )
        for c in range(max(0, NCHUNK - nbuf), NCHUNK):
            outs[c % nbuf].wait()

    return gather_k


def _mlp_body(g1, g2, ot, w1t, b1, w2, b2, w3, b3, obuf, out):
    del obuf
    w1full = w1t[...]
    w1at = w1full[:, :EMB_DIM]
    w1bt = w1full[:, EMB_DIM : 2 * EMB_DIM]
    w1ct = w1full[:, 2 * EMB_DIM :]
    dn_rt = (((1,), (1,)), ((), ()))
    dn_lt = (((0,), (0,)), ((), ()))
    hg = (
        lax.dot_general(g1[...], w1at, dn_rt, preferred_element_type=jnp.float32)
        + lax.dot_general(g2[...], w1bt, dn_rt, preferred_element_type=jnp.float32)
        + b1[...]
    )
    h1t = hg.T + jnp.dot(w1ct, ot[...], preferred_element_type=jnp.float32)
    h1t = jnp.maximum(h1t, 0.0)
    h2t = jnp.maximum(
        lax.dot_general(w2[...], h1t, dn_lt, preferred_element_type=jnp.float32)
        + b2[...].T,
        0.0,
    )
    out[...] = (
        lax.dot_general(w3[...], h2t, dn_lt, preferred_element_type=jnp.float32)
        + b3[...].T
    )


def _mlp_call(block0, cb, gathered, other_t, W1t, b1, W2, b2, W3, b3, obuf=None):
    nb = cb // BB
    full = lambda shape: pl.BlockSpec(shape, lambda i: (0, 0))
    in_specs = [
        pl.BlockSpec((BB, EMB_DIM), lambda i: (i, 0)),
        pl.BlockSpec((BB, EMB_DIM), lambda i: (i + nb, 0)),
        pl.BlockSpec((OTHER_DIM, BB), lambda i: (0, i + block0)),
        full((HIDDEN, IN1)),
        full((1, HIDDEN)),
        full((HIDDEN, HIDDEN)),
        full((1, HIDDEN)),
        full((HIDDEN, OUT_DIM)),
        full((1, OUT_DIM)),
    ]
    args = [gathered, gathered, other_t, W1t, b1, W2, b2, W3, b3]
    aliases = {}
    if obuf is not None:
        in_specs.append(pl.BlockSpec(memory_space=pl.ANY))
        args.append(obuf)
        aliases = {9: 0}
        body = _mlp_body
    else:

        def body(*refs):
            _mlp_body(*refs[:9], None, refs[9])

    return pl.pallas_call(
        body,
        grid=(nb,),
        in_specs=in_specs,
        out_specs=pl.BlockSpec((OUT_DIM, BB), lambda i: (0, i + block0)),
        out_shape=jax.ShapeDtypeStruct((OUT_DIM, BATCH), jnp.float32),
        input_output_aliases=aliases,
    )(*args)


def kernel(x, emb, W1, b1, W2, b2, W3, b3):
    x_t = x.T
    other_t = x_t[2:]

    W1t = W1.T
    b1r = b1.reshape(1, HIDDEN)
    b2r = b2.reshape(1, HIDDEN)
    b3r = b3.reshape(1, OUT_DIM)

    obuf = None
    start = 0
    for cb in CHUNKS:
        gathered = _make_sc_gather(start, cb)(emb, x_t)
        obuf = _mlp_call(
            start // BB, cb, gathered, other_t, W1t, b1r, W2, b2r, W3, b3r, obuf
        )
        start += cb
    return obuf.T
